# baseline (speedup 1.0000x reference)
import functools

import jax
import jax.numpy as jnp
from jax import lax
from jax.experimental import pallas as pl
from jax.experimental.pallas import tpu as pltpu
from jax.experimental.pallas import tpu_sc as plsc

N_NODES = 10000
D = 128
NC = 2
NS = 16
NW = NC * NS
CHUNK = 125
SUP = 16
PAD_ROWS = 240
N_PAD = N_NODES + PAD_ROWS
ROWS_PER_TILE = N_PAD // NS
ICHUNK = 80


def _deg_body(dsts_hbm, deg_out, idx_v, ones_v, buf_v, hist_sh):
    c = lax.axis_index("c")
    s = lax.axis_index("s")
    w = s * NC + c
    cpt = dsts_hbm.shape[1]

    pltpu.sync_copy(dsts_hbm.at[w], idx_v)

    def _fill(i, _):
        ones_v[pl.ds(i * 16, 16)] = jnp.full((16,), 1.0, jnp.float32)
        return 0
    lax.fori_loop(0, ones_v.shape[0] // 16, _fill, 0)

    def _zero(i, _):
        buf_v[pl.ds(i * 16, 16)] = jnp.zeros((16,), jnp.float32)
        return 0
    lax.fori_loop(0, ROWS_PER_TILE // 16, _zero, 0)

    base = s * ROWS_PER_TILE
    pltpu.sync_copy(buf_v, hist_sh.at[pl.ds(base, ROWS_PER_TILE)])
    plsc.subcore_barrier()

    def _step(j, _):
        pltpu.sync_copy(ones_v.at[pl.ds(0, CHUNK)], hist_sh.at[idx_v.at[j]],
                        add=True)
        return 0
    lax.fori_loop(0, cpt, _step, 0)
    plsc.subcore_barrier()

    pltpu.sync_copy(hist_sh.at[pl.ds(base, ROWS_PER_TILE)], buf_v)
    pltpu.sync_copy(buf_v, deg_out.at[c, pl.ds(base, ROWS_PER_TILE)])


def _msg_body(h_hbm, srcs_hbm, dsts_hbm, out_hbm, idx_s_v, idx_d_v, rows_v,
              acc_sh, sem0, sem1, sem_ia, sem_ib):
    c = lax.axis_index("c")
    s = lax.axis_index("s")
    w = s * NC + c
    cpt = srcs_hbm.shape[1]
    nsup = cpt // SUP
    rows0_v = rows_v.at[0]
    rows1_v = rows_v.at[1]
    base = s * ROWS_PER_TILE

    pltpu.async_copy(srcs_hbm.at[w, pl.ds(0, SUP)], idx_s_v.at[0], sem_ia)
    pltpu.async_copy(dsts_hbm.at[w, pl.ds(0, SUP)], idx_d_v.at[0], sem_ia)

    ibuf0 = rows0_v.at[pl.ds(0, ICHUNK)]
    ibuf1 = rows1_v.at[pl.ds(0, ICHUNK)]
    n_init = ROWS_PER_TILE // ICHUNK

    @pl.when(c == 0)
    def _():
        pltpu.async_copy(h_hbm.at[pl.ds(base, ICHUNK)], ibuf0, sem0)
        for k in range(n_init):
            ib = ibuf0 if k % 2 == 0 else ibuf1
            sm = sem0 if k % 2 == 0 else sem1
            pltpu.make_async_copy(
                h_hbm.at[pl.ds(base + k * ICHUNK, ICHUNK)], ib, sm).wait()
            if k + 1 < n_init:
                nib = ibuf1 if k % 2 == 0 else ibuf0
                nsm = sem1 if k % 2 == 0 else sem0
                pltpu.async_copy(
                    h_hbm.at[pl.ds(base + (k + 1) * ICHUNK, ICHUNK)], nib, nsm)
            pltpu.sync_copy(ib, acc_sh.at[pl.ds(base + k * ICHUNK, ICHUNK)])

    @pl.when(c == 1)
    def _():
        def _zero(i, _):
            r = i // (D // 16)
            col = (i % (D // 16)) * 16
            rows0_v[r, pl.ds(col, 16)] = jnp.zeros((16,), jnp.float32)
            return 0
        lax.fori_loop(0, ICHUNK * D // 16, _zero, 0)
        for k in range(n_init):
            pltpu.sync_copy(ibuf0, acc_sh.at[pl.ds(base + k * ICHUNK, ICHUNK)])

    plsc.subcore_barrier()

    def _wait_idx(sci, sm):
        sup0 = sci * SUP
        b = sci % 2
        pltpu.make_async_copy(
            srcs_hbm.at[w, pl.ds(sup0, SUP)], idx_s_v.at[b], sm).wait()
        pltpu.make_async_copy(
            dsts_hbm.at[w, pl.ds(sup0, SUP)], idx_d_v.at[b], sm).wait()

    def _fire_idx(sci, sm):
        sup0 = sci * SUP
        b = sci % 2
        pltpu.async_copy(srcs_hbm.at[w, pl.ds(sup0, SUP)], idx_s_v.at[b], sm)
        pltpu.async_copy(dsts_hbm.at[w, pl.ds(sup0, SUP)], idx_d_v.at[b], sm)

    def _gather(j, rbuf, sm):
        b = (j // SUP) % 2
        k = j % SUP
        return pltpu.async_copy(h_hbm.at[idx_s_v.at[b, k]], rbuf, sm)

    def _gather_wait(j, rbuf, sm):
        b = (j // SUP) % 2
        k = j % SUP
        pltpu.make_async_copy(h_hbm.at[idx_s_v.at[b, k]], rbuf, sm).wait()

    def _scatter(j, rbuf):
        b = (j // SUP) % 2
        k = j % SUP
        pltpu.sync_copy(rbuf, acc_sh.at[idx_d_v.at[b, k]], add=True)

    if nsup > 1:
        _fire_idx(1, sem_ib)
    _wait_idx(0, sem_ia)
    _gather(0, rows0_v, sem0)
    _gather(1, rows1_v, sem1)

    def _step(i, _):
        j0 = 2 * i
        j1 = 2 * i + 1
        _gather_wait(j0, rows0_v, sem0)
        _scatter(j0, rows0_v)

        sci_w = j0 // SUP + 1
        cond_w = (j0 % SUP == SUP - 2) & (j0 + 2 < cpt)

        @pl.when(cond_w & (sci_w % 2 == 0))
        def _():
            _wait_idx(sci_w, sem_ia)

        @pl.when(cond_w & (sci_w % 2 == 1))
        def _():
            _wait_idx(sci_w, sem_ib)

        @pl.when(j0 + 2 < cpt)
        def _():
            _gather(j0 + 2, rows0_v, sem0)

        _gather_wait(j1, rows1_v, sem1)
        _scatter(j1, rows1_v)

        sci_f = j1 // SUP + 2
        cond_f = (j1 % SUP == SUP - 1) & (sci_f < nsup)

        @pl.when(cond_f & (sci_f % 2 == 0))
        def _():
            _fire_idx(sci_f, sem_ia)

        @pl.when(cond_f & (sci_f % 2 == 1))
        def _():
            _fire_idx(sci_f, sem_ib)

        @pl.when(j1 + 2 < cpt)
        def _():
            _gather(j1 + 2, rows1_v, sem1)
        return 0
    lax.fori_loop(0, cpt // 2, _step, 0)
    plsc.subcore_barrier()

    for k in range(ROWS_PER_TILE // ICHUNK):
        ib = ibuf0 if k % 2 == 0 else ibuf1
        sm = sem0 if k % 2 == 0 else sem1
        if k >= 2:
            pltpu.make_async_copy(
                ib, out_hbm.at[c, pl.ds(base + (k - 2) * ICHUNK, ICHUNK)],
                sm).wait()
        pltpu.sync_copy(acc_sh.at[pl.ds(base + k * ICHUNK, ICHUNK)], ib)
        pltpu.async_copy(ib, out_hbm.at[c, pl.ds(base + k * ICHUNK, ICHUNK)],
                         sm)
    n_wb = ROWS_PER_TILE // ICHUNK
    for k in range(max(n_wb - 2, 0), n_wb):
        ib = ibuf0 if k % 2 == 0 else ibuf1
        sm = sem0 if k % 2 == 0 else sem1
        pltpu.make_async_copy(
            ib, out_hbm.at[c, pl.ds(base + k * ICHUNK, ICHUNK)], sm).wait()


def _make_sc_kernels(cpt):
    mesh = plsc.VectorSubcoreMesh(core_axis_name="c", subcore_axis_name="s")
    deg = pl.kernel(
        _deg_body,
        out_type=jax.ShapeDtypeStruct((NC, N_PAD), jnp.float32),
        mesh=mesh,
        scratch_types=[
            pltpu.VMEM((cpt, CHUNK), jnp.int32),
            pltpu.VMEM((128,), jnp.float32),
            pltpu.VMEM((ROWS_PER_TILE,), jnp.float32),
            pltpu.VMEM_SHARED((N_PAD,), jnp.float32),
        ],
    )
    msg = pl.kernel(
        _msg_body,
        out_type=jax.ShapeDtypeStruct((NC, N_PAD, D), jnp.float32),
        mesh=mesh,
        scratch_types=[
            pltpu.VMEM((2, SUP, CHUNK), jnp.int32),
            pltpu.VMEM((2, SUP, CHUNK), jnp.int32),
            pltpu.VMEM((2, CHUNK, D), jnp.float32),
            pltpu.VMEM_SHARED((N_PAD, D), jnp.float32),
            pltpu.SemaphoreType.DMA,
            pltpu.SemaphoreType.DMA,
            pltpu.SemaphoreType.DMA,
            pltpu.SemaphoreType.DMA,
        ],
    )
    return deg, msg


_BROWS = 512


def _dinv(degT_ref):
    d = degT_ref[:, 0:1] + degT_ref[:, 1:2] + 1.0
    return lax.rsqrt(d)


def _prep_body(x_ref, w_ref, degT_ref, o_ref):
    o_ref[...] = (
        jnp.dot(x_ref[...], w_ref[...], preferred_element_type=jnp.float32)
        * _dinv(degT_ref)
    )


def _mid_body(a_ref, degT_ref, b_ref, w_ref, o_ref):
    dinv = _dinv(degT_ref)
    z = jnp.maximum((a_ref[0] + a_ref[1]) * dinv + b_ref[...], 0.0)
    o_ref[...] = (
        jnp.dot(z, w_ref[...], preferred_element_type=jnp.float32) * dinv
    )


def _final_body(a_ref, degT_ref, b_ref, o_ref):
    o_ref[...] = (a_ref[0] + a_ref[1]) * _dinv(degT_ref) + b_ref[...]


def _row_spec(b, d):
    return pl.BlockSpec((b, d), lambda i: (i, 0))


_W_SPEC = pl.BlockSpec((D, D), lambda i: (0, 0))
_B_SPEC = pl.BlockSpec((1, D), lambda i: (0, 0))
_ACC_SPEC = pl.BlockSpec((NC, _BROWS, D), lambda i: (0, i, 0))
_GRID = (N_PAD // _BROWS,)
_HD = jax.ShapeDtypeStruct((N_PAD, D), jnp.float32)

_prep = pl.pallas_call(
    _prep_body, grid=_GRID,
    in_specs=[_row_spec(_BROWS, D), _W_SPEC, _row_spec(_BROWS, NC)],
    out_specs=_row_spec(_BROWS, D), out_shape=_HD)

_mid = pl.pallas_call(
    _mid_body, grid=_GRID,
    in_specs=[_ACC_SPEC, _row_spec(_BROWS, NC), _B_SPEC, _W_SPEC],
    out_specs=_row_spec(_BROWS, D), out_shape=_HD)

_BF = 1000
_final = pl.pallas_call(
    _final_body, grid=(N_NODES // _BF,),
    in_specs=[pl.BlockSpec((NC, _BF, D), lambda i: (0, i, 0)),
              _row_spec(_BF, NC), _B_SPEC],
    out_specs=_row_spec(_BF, D),
    out_shape=jax.ShapeDtypeStruct((N_NODES, D), jnp.float32))


@jax.jit
def kernel(x, edge_index, W1, b1, W2, b2):
    E = edge_index.shape[1]
    cpt = -(-E // (NW * CHUNK))
    cpt = -(-cpt // SUP) * SUP
    e_pad = cpt * NW * CHUNK
    npad = e_pad - E

    src = edge_index[0].astype(jnp.int32)
    dst = edge_index[1].astype(jnp.int32)
    if npad:
        pad_idx = N_NODES + (jnp.arange(npad, dtype=jnp.int32) % PAD_ROWS)
        src = jnp.concatenate([src, pad_idx])
        dst = jnp.concatenate([dst, pad_idx])
    srcs = src.reshape(NW, cpt, CHUNK)
    dsts = dst.reshape(NW, cpt, CHUNK)

    b1r = b1.reshape(1, D)
    b2r = b2.reshape(1, D)

    deg_k, msg_k = _make_sc_kernels(cpt)

    deg_parts = deg_k(dsts)
    degT = deg_parts.T.reshape(N_PAD, NC)

    h1p = _prep(x, W1, degT)
    acc1 = msg_k(h1p, srcs, dsts)
    h2p = _mid(acc1, degT, b1r, W2)
    acc2 = msg_k(h2p, srcs, dsts)
    return _final(acc2, degT, b2r)


# speedup vs baseline: 1.1225x; 1.1225x over previous
import functools

import jax
import jax.numpy as jnp
from jax import lax
from jax.experimental import pallas as pl
from jax.experimental.pallas import tpu as pltpu
from jax.experimental.pallas import tpu_sc as plsc

N_NODES = 10000
D = 128
NC = 2
NS = 16
NW = NC * NS
CHUNK = 128
SUP = 16
N_PAD = 10240
ROWS_PER_TILE = N_PAD // NS
ICHUNK = 128


def _deg_body(ei3_hbm, deg_out, idx_v, ones_v, buf_v, hist_sh):
    c = lax.axis_index("c")
    s = lax.axis_index("s")
    w = s * NC + c
    cpt = idx_v.shape[0]
    nblocks = ei3_hbm.shape[0]
    n_tail = nblocks - cpt * NW

    pltpu.sync_copy(ei3_hbm.at[pl.ds(w * cpt, cpt)], idx_v)

    def _fill(i, _):
        ones_v[pl.ds(i * 16, 16)] = jnp.full((16,), 1.0, jnp.float32)
        return 0
    lax.fori_loop(0, ones_v.shape[0] // 16, _fill, 0)

    def _zero(i, _):
        buf_v[pl.ds(i * 16, 16)] = jnp.zeros((16,), jnp.float32)
        return 0
    lax.fori_loop(0, ROWS_PER_TILE // 16, _zero, 0)

    base = s * ROWS_PER_TILE
    pltpu.sync_copy(buf_v, hist_sh.at[pl.ds(base, ROWS_PER_TILE)])
    plsc.subcore_barrier()

    def _step(j, _):
        pltpu.sync_copy(ones_v.at[pl.ds(0, CHUNK)],
                        hist_sh.at[idx_v.at[j, 1]], add=True)
        return 0
    lax.fori_loop(0, cpt, _step, 0)

    if n_tail:
        @pl.when(w < n_tail)
        def _():
            pltpu.sync_copy(ei3_hbm.at[cpt * NW + w, 1], idx_v.at[0, 0])
            pltpu.sync_copy(ones_v.at[pl.ds(0, CHUNK)],
                            hist_sh.at[idx_v.at[0, 0]], add=True)
    plsc.subcore_barrier()

    pltpu.sync_copy(hist_sh.at[pl.ds(base, ROWS_PER_TILE)], buf_v)
    pltpu.sync_copy(buf_v, deg_out.at[c, pl.ds(base, ROWS_PER_TILE)])


def _msg_body(h_hbm, ei3_hbm, out_hbm, idx_s_v, idx_d_v, rows_v,
              acc_sh, sem0, sem1, sem_ia, sem_ib):
    c = lax.axis_index("c")
    s = lax.axis_index("s")
    w = s * NC + c
    nblocks = ei3_hbm.shape[0]
    cpt = nblocks // NW
    n_tail = nblocks - cpt * NW
    nsup = -(-cpt // SUP)
    rows0_v = rows_v.at[0]
    rows1_v = rows_v.at[1]
    base = s * ROWS_PER_TILE

    pltpu.async_copy(ei3_hbm.at[pl.ds(w * cpt, SUP), 0], idx_s_v.at[0],
                     sem_ia)
    pltpu.async_copy(ei3_hbm.at[pl.ds(w * cpt, SUP), 1], idx_d_v.at[0],
                     sem_ia)

    ibuf0 = rows0_v.at[pl.ds(0, ICHUNK)]
    ibuf1 = rows1_v.at[pl.ds(0, ICHUNK)]
    n_init = ROWS_PER_TILE // ICHUNK

    @pl.when(c == 0)
    def _():
        pltpu.async_copy(h_hbm.at[pl.ds(base, ICHUNK)], ibuf0, sem0)
        for k in range(n_init):
            ib = ibuf0 if k % 2 == 0 else ibuf1
            sm = sem0 if k % 2 == 0 else sem1
            pltpu.make_async_copy(
                h_hbm.at[pl.ds(base + k * ICHUNK, ICHUNK)], ib, sm).wait()
            if k + 1 < n_init:
                nib = ibuf1 if k % 2 == 0 else ibuf0
                nsm = sem1 if k % 2 == 0 else sem0
                pltpu.async_copy(
                    h_hbm.at[pl.ds(base + (k + 1) * ICHUNK, ICHUNK)], nib, nsm)
            pltpu.sync_copy(ib, acc_sh.at[pl.ds(base + k * ICHUNK, ICHUNK)])

    @pl.when(c == 1)
    def _():
        def _zero(r, _):
            for g in range(D // 16):
                rows0_v[r, pl.ds(g * 16, 16)] = jnp.zeros((16,), jnp.float32)
            return 0
        lax.fori_loop(0, ICHUNK, _zero, 0)
        for k in range(n_init):
            pltpu.sync_copy(ibuf0, acc_sh.at[pl.ds(base + k * ICHUNK, ICHUNK)])

    plsc.subcore_barrier()

    def _wait_idx(sci, sm):
        blk0 = w * cpt + sci * SUP
        b = sci % 2
        pltpu.make_async_copy(
            ei3_hbm.at[pl.ds(blk0, SUP), 0], idx_s_v.at[b], sm).wait()
        pltpu.make_async_copy(
            ei3_hbm.at[pl.ds(blk0, SUP), 1], idx_d_v.at[b], sm).wait()

    def _fire_idx(sci, sm):
        blk0 = w * cpt + sci * SUP
        b = sci % 2
        pltpu.async_copy(ei3_hbm.at[pl.ds(blk0, SUP), 0], idx_s_v.at[b], sm)
        pltpu.async_copy(ei3_hbm.at[pl.ds(blk0, SUP), 1], idx_d_v.at[b], sm)

    def _gather(j, rbuf, sm):
        b = (j // SUP) % 2
        k = j % SUP
        return pltpu.async_copy(h_hbm.at[idx_s_v.at[b, k]], rbuf, sm)

    def _gather_wait(j, rbuf, sm):
        b = (j // SUP) % 2
        k = j % SUP
        pltpu.make_async_copy(h_hbm.at[idx_s_v.at[b, k]], rbuf, sm).wait()

    def _scatter(j, rbuf):
        b = (j // SUP) % 2
        k = j % SUP
        pltpu.sync_copy(rbuf, acc_sh.at[idx_d_v.at[b, k]], add=True)

    if nsup > 1:
        _fire_idx(1, sem_ib)
    _wait_idx(0, sem_ia)
    _gather(0, rows0_v, sem0)
    _gather(1, rows1_v, sem1)

    def _step(i, _):
        j0 = 2 * i
        j1 = 2 * i + 1
        _gather_wait(j0, rows0_v, sem0)
        _scatter(j0, rows0_v)

        sci_w = j0 // SUP + 1
        cond_w = (j0 % SUP == SUP - 2) & (j0 + 2 < cpt)

        @pl.when(cond_w & (sci_w % 2 == 0))
        def _():
            _wait_idx(sci_w, sem_ia)

        @pl.when(cond_w & (sci_w % 2 == 1))
        def _():
            _wait_idx(sci_w, sem_ib)

        @pl.when(j0 + 2 < cpt)
        def _():
            _gather(j0 + 2, rows0_v, sem0)

        _gather_wait(j1, rows1_v, sem1)
        _scatter(j1, rows1_v)

        sci_f = j1 // SUP + 2
        cond_f = (j1 % SUP == SUP - 1) & (sci_f < nsup)

        @pl.when(cond_f & (sci_f % 2 == 0))
        def _():
            _fire_idx(sci_f, sem_ia)

        @pl.when(cond_f & (sci_f % 2 == 1))
        def _():
            _fire_idx(sci_f, sem_ib)

        @pl.when(j1 + 2 < cpt)
        def _():
            _gather(j1 + 2, rows1_v, sem1)
        return 0
    lax.fori_loop(0, cpt // 2, _step, 0)

    if n_tail:
        @pl.when(w < n_tail)
        def _():
            blk = cpt * NW + w
            pltpu.sync_copy(ei3_hbm.at[blk, 0], idx_s_v.at[0, 0])
            pltpu.sync_copy(ei3_hbm.at[blk, 1], idx_d_v.at[0, 0])
            pltpu.sync_copy(h_hbm.at[idx_s_v.at[0, 0]], rows0_v)
            pltpu.sync_copy(rows0_v, acc_sh.at[idx_d_v.at[0, 0]], add=True)
    plsc.subcore_barrier()

    for k in range(ROWS_PER_TILE // ICHUNK):
        ib = ibuf0 if k % 2 == 0 else ibuf1
        sm = sem0 if k % 2 == 0 else sem1
        if k >= 2:
            pltpu.make_async_copy(
                ib, out_hbm.at[c, pl.ds(base + (k - 2) * ICHUNK, ICHUNK)],
                sm).wait()
        pltpu.sync_copy(acc_sh.at[pl.ds(base + k * ICHUNK, ICHUNK)], ib)
        pltpu.async_copy(ib, out_hbm.at[c, pl.ds(base + k * ICHUNK, ICHUNK)],
                         sm)
    n_wb = ROWS_PER_TILE // ICHUNK
    for k in range(max(n_wb - 2, 0), n_wb):
        ib = ibuf0 if k % 2 == 0 else ibuf1
        sm = sem0 if k % 2 == 0 else sem1
        pltpu.make_async_copy(
            ib, out_hbm.at[c, pl.ds(base + k * ICHUNK, ICHUNK)], sm).wait()


def _make_sc_kernels(cpt):
    mesh = plsc.VectorSubcoreMesh(core_axis_name="c", subcore_axis_name="s")
    deg = pl.kernel(
        _deg_body,
        out_type=jax.ShapeDtypeStruct((NC, N_PAD), jnp.float32),
        mesh=mesh,
        scratch_types=[
            pltpu.VMEM((cpt, 2, CHUNK), jnp.int32),
            pltpu.VMEM((128,), jnp.float32),
            pltpu.VMEM((ROWS_PER_TILE,), jnp.float32),
            pltpu.VMEM_SHARED((N_PAD,), jnp.float32),
        ],
    )
    msg = pl.kernel(
        _msg_body,
        out_type=jax.ShapeDtypeStruct((NC, N_PAD, D), jnp.float32),
        mesh=mesh,
        scratch_types=[
            pltpu.VMEM((2, SUP, CHUNK), jnp.int32),
            pltpu.VMEM((2, SUP, CHUNK), jnp.int32),
            pltpu.VMEM((2, CHUNK, D), jnp.float32),
            pltpu.VMEM_SHARED((N_PAD, D), jnp.float32),
            pltpu.SemaphoreType.DMA,
            pltpu.SemaphoreType.DMA,
            pltpu.SemaphoreType.DMA,
            pltpu.SemaphoreType.DMA,
        ],
    )
    return deg, msg


_BROWS = 2048


def _dinv(deg_ref):
    d = deg_ref[0:1, :] + deg_ref[1:2, :] + 1.0
    return lax.rsqrt(jnp.transpose(d, (1, 0)))


def _prep_body(x_ref, w_ref, deg_ref, o_ref):
    o_ref[...] = (
        jnp.dot(x_ref[...], w_ref[...], preferred_element_type=jnp.float32)
        * _dinv(deg_ref)
    )


def _mid_body(a_ref, deg_ref, b_ref, w_ref, o_ref):
    dinv = _dinv(deg_ref)
    z = jnp.maximum((a_ref[0] + a_ref[1]) * dinv + b_ref[...], 0.0)
    o_ref[...] = (
        jnp.dot(z, w_ref[...], preferred_element_type=jnp.float32) * dinv
    )


def _final_body(a_ref, deg_ref, b_ref, o_ref):
    o_ref[...] = (a_ref[0] + a_ref[1]) * _dinv(deg_ref) + b_ref[...]


def _row_spec(b, d):
    return pl.BlockSpec((b, d), lambda i: (i, 0))


_W_SPEC = pl.BlockSpec((D, D), lambda i: (0, 0))
_B_SPEC = pl.BlockSpec((1, D), lambda i: (0, 0))
_ACC_SPEC = pl.BlockSpec((NC, _BROWS, D), lambda i: (0, i, 0))
_GRID = (N_PAD // _BROWS,)
_HD = jax.ShapeDtypeStruct((N_PAD, D), jnp.float32)

_DEG_SPEC = pl.BlockSpec((NC, _BROWS), lambda i: (0, i))

_prep = pl.pallas_call(
    _prep_body, grid=_GRID,
    in_specs=[_row_spec(_BROWS, D), _W_SPEC, _DEG_SPEC],
    out_specs=_row_spec(_BROWS, D), out_shape=_HD)

_mid = pl.pallas_call(
    _mid_body, grid=_GRID,
    in_specs=[_ACC_SPEC, _DEG_SPEC, _B_SPEC, _W_SPEC],
    out_specs=_row_spec(_BROWS, D), out_shape=_HD)

_BF = 1024
_final = pl.pallas_call(
    _final_body, grid=(-(-N_NODES // _BF),),
    in_specs=[pl.BlockSpec((NC, _BF, D), lambda i: (0, i, 0)),
              pl.BlockSpec((NC, _BF), lambda i: (0, i)), _B_SPEC],
    out_specs=_row_spec(_BF, D),
    out_shape=jax.ShapeDtypeStruct((N_NODES, D), jnp.float32))


@jax.jit
def kernel(x, edge_index, W1, b1, W2, b2):
    E = edge_index.shape[1]
    nblocks = E // CHUNK
    cpt = nblocks // NW

    ei3 = jnp.transpose(
        edge_index.astype(jnp.int32).reshape(2, nblocks, CHUNK), (1, 0, 2))

    b1r = b1.reshape(1, D)
    b2r = b2.reshape(1, D)

    deg_k, msg_k = _make_sc_kernels(cpt)

    deg_parts = deg_k(ei3)

    h1p = _prep(x, W1, deg_parts)
    acc1 = msg_k(h1p, ei3)
    h2p = _mid(acc1, deg_parts, b1r, W2)
    acc2 = msg_k(h2p, ei3)
    return _final(acc2, deg_parts, b2r)
